# baseline (speedup 1.0000x reference)
import functools

import jax
import jax.numpy as jnp
from jax.experimental import pallas as pl

RATIO = 16


def _body(mask_ref, emb_ref, out_ref, flag_ref):
    flags = jnp.all(mask_ref[...], axis=2)
    flag_ref[...] = flags
    keep = 1.0 - flags.astype(jnp.float32)
    out_ref[...] = keep[:, :, None] * emb_ref[1, :][None, None, :]


@functools.lru_cache(maxsize=None)
def _make_tc_call(B: int, Sl: int, d: int, grid: int):
    Sg = Sl // RATIO
    sblk = Sg // grid
    return pl.pallas_call(
        _body,
        grid=(grid,),
        in_specs=[
            pl.BlockSpec((B, sblk, RATIO), lambda i: (0, i, 0)),
            pl.BlockSpec((2, d), lambda i: (0, 0)),
        ],
        out_specs=[
            pl.BlockSpec((B, sblk, d), lambda i: (0, i, 0)),
            pl.BlockSpec((B, sblk), lambda i: (0, i)),
        ],
        out_shape=[
            jax.ShapeDtypeStruct((B, Sg, d), jnp.float32),
            jax.ShapeDtypeStruct((B, Sg), jnp.bool_),
        ],
    )


def kernel(token_ids, padding_mask, embeds):
    B, Sl = padding_mask.shape
    d = embeds.shape[1]
    Sg = Sl // RATIO
    return _make_tc_call(B, Sl, d, 4)(
        padding_mask.reshape(B, Sg, RATIO), embeds)


# speedup vs baseline: 1.0956x; 1.0956x over previous
import functools

import jax
import jax.numpy as jnp
from jax.experimental import pallas as pl

RATIO = 16


def _body(mask_ref, emb_ref, out_ref, flag_ref):
    flags = jnp.all(mask_ref[...], axis=2)
    flag_ref[...] = flags
    keep = 1.0 - flags.astype(jnp.float32)
    out_ref[...] = keep[:, :, None] * emb_ref[1, :][None, None, :]


@functools.lru_cache(maxsize=None)
def _make_tc_call(B: int, Sl: int, d: int, grid: int):
    Sg = Sl // RATIO
    sblk = Sg // grid
    return pl.pallas_call(
        _body,
        grid=(grid,),
        in_specs=[
            pl.BlockSpec((B, sblk, RATIO), lambda i: (0, i, 0)),
            pl.BlockSpec((2, d), lambda i: (0, 0)),
        ],
        out_specs=[
            pl.BlockSpec((B, sblk, d), lambda i: (0, i, 0)),
            pl.BlockSpec((B, sblk), lambda i: (0, i)),
        ],
        out_shape=[
            jax.ShapeDtypeStruct((B, Sg, d), jnp.float32),
            jax.ShapeDtypeStruct((B, Sg), jnp.bool_),
        ],
    )


def kernel(token_ids, padding_mask, embeds):
    B, Sl = padding_mask.shape
    d = embeds.shape[1]
    Sg = Sl // RATIO
    return _make_tc_call(B, Sl, d, 2)(
        padding_mask.reshape(B, Sg, RATIO), embeds)


# speedup vs baseline: 1.1588x; 1.0577x over previous
import functools

import jax
import jax.numpy as jnp
from jax.experimental import pallas as pl

RATIO = 16
LANES = 128


@functools.lru_cache(maxsize=None)
def _make_tc_call(B: int, Sl: int, d: int, grid: int):
    nsub = Sl // LANES
    gpl = LANES // RATIO
    dblk = d // grid

    def body(mask_ref, emb_ref, out_ref, flag_ref):
        m = mask_ref[...].astype(jnp.float32).reshape(B * nsub, LANES)
        sel = (jax.lax.broadcasted_iota(jnp.int32, (LANES, gpl), 0) // RATIO
               == jax.lax.broadcasted_iota(jnp.int32, (LANES, gpl), 1)
               ).astype(jnp.float32)
        sums = jax.lax.dot_general(
            m, sel, (((1,), (0,)), ((), ())),
            preferred_element_type=jnp.float32)
        allset = sums == float(RATIO)
        flag_ref[...] = jnp.where(allset, 1, 0).reshape(B, nsub, gpl)
        keep = 1.0 - allset.astype(jnp.float32)
        out_ref[...] = (keep.reshape(B, nsub, gpl)[:, :, :, None]
                        * emb_ref[1, :][None, None, None, :])

    return pl.pallas_call(
        body,
        grid=(grid,),
        in_specs=[
            pl.BlockSpec((B, Sl), lambda i: (0, 0)),
            pl.BlockSpec((2, dblk), lambda i: (0, i)),
        ],
        out_specs=[
            pl.BlockSpec((B, nsub, gpl, dblk), lambda i: (0, 0, 0, i)),
            pl.BlockSpec((B, nsub, gpl), lambda i: (0, 0, 0)),
        ],
        out_shape=[
            jax.ShapeDtypeStruct((B, nsub, gpl, d), jnp.float32),
            jax.ShapeDtypeStruct((B, nsub, gpl), jnp.int32),
        ],
    )


def kernel(token_ids, padding_mask, embeds):
    B, Sl = padding_mask.shape
    d = embeds.shape[1]
    Sg = Sl // RATIO
    out4, flags = _make_tc_call(B, Sl, d, 2)(padding_mask, embeds)
    return (out4.reshape(B, Sg, d),
            (flags != 0).reshape(B, Sg))


# speedup vs baseline: 1.2040x; 1.0390x over previous
import functools

import jax
import jax.numpy as jnp
from jax.experimental import pallas as pl

RATIO = 16
LANES = 128


@functools.lru_cache(maxsize=None)
def _make_tc_call(B: int, Sl: int, d: int, grid: int):
    nsub = Sl // LANES
    gpl = LANES // RATIO
    dblk = d // grid

    Sg = Sl // RATIO

    def body(mask_ref, emb_ref, out_ref, flag_ref):
        m = mask_ref[...].astype(jnp.float32).reshape(B * nsub, LANES)
        sel = (jax.lax.broadcasted_iota(jnp.int32, (LANES, gpl), 0) // RATIO
               == jax.lax.broadcasted_iota(jnp.int32, (LANES, gpl), 1)
               ).astype(jnp.float32)
        sums = jax.lax.dot_general(
            m, sel, (((1,), (0,)), ((), ())),
            preferred_element_type=jnp.float32)
        allset = sums == float(RATIO)
        keep = 1.0 - allset.astype(jnp.float32)
        out_ref[...] = (keep.reshape(B, nsub, gpl)[:, :, :, None]
                        * emb_ref[1, :][None, None, None, :])
        rep = (jax.lax.broadcasted_iota(jnp.int32, (gpl, Sg), 0)
               == jax.lax.broadcasted_iota(
                   jnp.int32, (gpl, Sg), 1) % gpl).astype(jnp.float32)
        wide = jax.lax.dot_general(
            sums, rep, (((1,), (0,)), ((), ())),
            preferred_element_type=jnp.float32)
        pick = (jax.lax.broadcasted_iota(jnp.int32, (nsub, Sg), 0)
                == jax.lax.broadcasted_iota(
                    jnp.int32, (nsub, Sg), 1) // gpl).astype(jnp.float32)
        flags2d = jnp.sum(wide.reshape(B, nsub, Sg) * pick[None],
                          axis=1)
        flag_ref[...] = jnp.where(flags2d == float(RATIO), 1, 0)

    return pl.pallas_call(
        body,
        grid=(grid,),
        in_specs=[
            pl.BlockSpec((B, Sl), lambda i: (0, 0)),
            pl.BlockSpec((2, dblk), lambda i: (0, i)),
        ],
        out_specs=[
            pl.BlockSpec((B, nsub, gpl, dblk), lambda i: (0, 0, 0, i)),
            pl.BlockSpec((B, Sg), lambda i: (0, 0)),
        ],
        out_shape=[
            jax.ShapeDtypeStruct((B, nsub, gpl, d), jnp.float32),
            jax.ShapeDtypeStruct((B, Sg), jnp.int32),
        ],
    )


def kernel(token_ids, padding_mask, embeds):
    B, Sl = padding_mask.shape
    d = embeds.shape[1]
    Sg = Sl // RATIO
    out4, flags = _make_tc_call(B, Sl, d, 2)(padding_mask, embeds)
    return out4.reshape(B, Sg, d), flags != 0


# speedup vs baseline: 1.2219x; 1.0149x over previous
import functools

import jax
import jax.numpy as jnp
from jax.experimental import pallas as pl

RATIO = 16
LANES = 128


@functools.lru_cache(maxsize=None)
def _make_tc_call(B: int, Sl: int, d: int, grid: int):
    nsub = Sl // LANES
    gpl = LANES // RATIO
    dblk = d // grid

    Sg = Sl // RATIO

    def body(mask_ref, emb_ref, out_ref, flag_ref):
        m = mask_ref[...].astype(jnp.float32).reshape(B * nsub, LANES)
        sel = (jax.lax.broadcasted_iota(jnp.int32, (LANES, gpl), 0) // RATIO
               == jax.lax.broadcasted_iota(jnp.int32, (LANES, gpl), 1)
               ).astype(jnp.float32)
        sums = jax.lax.dot_general(
            m, sel, (((1,), (0,)), ((), ())),
            preferred_element_type=jnp.float32)
        allset = sums == float(RATIO)
        keep = 1.0 - allset.astype(jnp.float32)
        out_ref[...] = (keep.reshape(B, nsub, gpl)[:, :, :, None]
                        * emb_ref[1, :][None, None, None, :])
        rep = (jax.lax.broadcasted_iota(jnp.int32, (gpl, Sg), 0)
               == jax.lax.broadcasted_iota(
                   jnp.int32, (gpl, Sg), 1) % gpl).astype(jnp.float32)
        wide = jax.lax.dot_general(
            sums, rep, (((1,), (0,)), ((), ())),
            preferred_element_type=jnp.float32)
        pick = (jax.lax.broadcasted_iota(jnp.int32, (nsub, Sg), 0)
                == jax.lax.broadcasted_iota(
                    jnp.int32, (nsub, Sg), 1) // gpl).astype(jnp.float32)
        flags2d = jnp.sum(wide.reshape(B, nsub, Sg) * pick[None],
                          axis=1)
        flag_ref[...] = jnp.where(flags2d == float(RATIO), 1, 0)

    return pl.pallas_call(
        body,
        grid=(grid,),
        in_specs=[
            pl.BlockSpec((B, Sl), lambda i: (0, 0)),
            pl.BlockSpec((2, dblk), lambda i: (0, i)),
        ],
        out_specs=[
            pl.BlockSpec((B, nsub, gpl, dblk), lambda i: (0, 0, 0, i)),
            pl.BlockSpec((B, Sg), lambda i: (0, 0)),
        ],
        out_shape=[
            jax.ShapeDtypeStruct((B, nsub, gpl, d), jnp.float32),
            jax.ShapeDtypeStruct((B, Sg), jnp.int32),
        ],
    )


def kernel(token_ids, padding_mask, embeds):
    B, Sl = padding_mask.shape
    d = embeds.shape[1]
    Sg = Sl // RATIO
    out4, flags = _make_tc_call(B, Sl, d, 2)(
        padding_mask.astype(jnp.int8), embeds)
    return out4.reshape(B, Sg, d), flags != 0
